# baseline (speedup 1.0000x reference)
import jax
import jax.numpy as jnp
from jax import lax
from jax.experimental import pallas as pl
from jax.experimental.pallas import tpu as pltpu
from jax.experimental.pallas import tpu_sc as plsc

A_DIM = 26
N_CLASSES = 100000
BATCH = 4096

_NC = 2
_NS = 16
_L = 16

_RB = BATCH // 128
_UNROLL = 10
_CHUNKS = N_CLASSES // _L
_STEPS = _CHUNKS // _UNROLL

_LN2 = 0.6931471805599453


def _vlog(x):
    bits = lax.bitcast_convert_type(x, jnp.int32)
    e = ((bits >> 23) & 0xFF) - 127
    mant = lax.bitcast_convert_type(
        (bits & 0x7FFFFF) | jnp.int32(0x3F800000), jnp.float32
    )
    t = mant - 1.0
    y = e.astype(jnp.float32) * _LN2 + t * (1.0 - t * (0.5 - t * (1.0 / 3.0)))
    for _ in range(3):
        y = y + x * jnp.exp(-y) - 1.0
    return y


def _sc_body(logits_hbm, valt_hbm, out_hbm, row_v, idx_v, acc_v, sidx_v, shared):
    c = lax.axis_index("c")
    s = lax.axis_index("s")
    wid = s * _NC + c

    @pl.when(wid < A_DIM)
    def _work():
        pltpu.sync_copy(logits_hbm.at[wid], row_v)
        pltpu.sync_copy(valt_hbm.at[wid], idx_v)

        def mx(i, carry):
            a, b = carry
            for u in range(0, _UNROLL, 2):
                a = jnp.maximum(a, row_v[pl.ds((i * _UNROLL + u) * _L, _L)])
                b = jnp.maximum(b, row_v[pl.ds((i * _UNROLL + u + 1) * _L, _L)])
            return a, b

        ninf = jnp.full((_L,), -jnp.inf, jnp.float32)
        ma, mb = lax.fori_loop(0, _STEPS, mx, (ninf, ninf))
        m = jnp.max(jnp.maximum(ma, mb))

        def se(i, carry):
            a, b = carry
            for u in range(0, _UNROLL, 2):
                a = a + jnp.exp(row_v[pl.ds((i * _UNROLL + u) * _L, _L)] - m)
                b = b + jnp.exp(row_v[pl.ds((i * _UNROLL + u + 1) * _L, _L)] - m)
            return a, b

        zero = jnp.zeros((_L,), jnp.float32)
        sa, sb = lax.fori_loop(0, _STEPS, se, (zero, zero))
        ssum = jnp.sum(sa + sb)
        lse_vec = _vlog(jnp.full((_L,), ssum, jnp.float32)) + m

        def outer(r, carry):
            for k in range(128 // _L):
                idx = idx_v[r, pl.ds(k * _L, _L)]
                acc_v[r, pl.ds(k * _L, _L)] = (
                    plsc.load_gather(row_v, [idx]) - lse_vec
                )
            return carry

        lax.fori_loop(0, _RB, outer, 0)

    iota = lax.broadcasted_iota(jnp.int32, (_L,), 0)
    sidx_v[pl.ds(0, _L)] = iota
    sidx_v[pl.ds(_L, _L)] = iota + _L

    plsc.subcore_barrier()

    @pl.when(s == 0)
    def _seed():
        pltpu.sync_copy(acc_v, shared)

    plsc.subcore_barrier()

    @pl.when((s >= 1) & (wid < A_DIM))
    def _accum():
        pltpu.sync_copy(acc_v, shared.at[sidx_v], add=True)

    plsc.subcore_barrier()

    @pl.when(s == 0)
    def _out():
        pltpu.sync_copy(shared, out_hbm.at[c])


def _sc_gather(logits, valt):
    mesh = plsc.VectorSubcoreMesh(
        core_axis_name="c", subcore_axis_name="s", num_cores=_NC, num_subcores=_NS
    )
    f = pl.kernel(
        _sc_body,
        out_type=jax.ShapeDtypeStruct((_NC, _RB, 128), jnp.float32),
        mesh=mesh,
        scratch_types=[
            pltpu.VMEM((N_CLASSES,), jnp.float32),
            pltpu.VMEM((_RB, 128), jnp.int32),
            pltpu.VMEM((_RB, 128), jnp.float32),
            pltpu.VMEM((2 * _L,), jnp.int32),
            pltpu.VMEM_SHARED((_RB, 128), jnp.float32),
        ],
        compiler_params=pltpu.CompilerParams(needs_layout_passes=False),
    )
    return f(logits, valt)


def kernel(logits, value):
    valt = value.T.reshape(A_DIM, _RB, 128)
    parts = _sc_gather(logits, valt)
    return (parts[0] + parts[1]).reshape(BATCH)
